# baseline (speedup 1.0000x reference)
import functools
import jax
import jax.numpy as jnp
from jax.experimental import pallas as pl
from jax.experimental.pallas import tpu as pltpu

B, T, D = 64, 32, 2048
NRE = 7
DFF = 2048
BF = 512
BD = 512
KF = DFF // BF
KD = D // BD


def _ffn1_body(eid_ref, perm_ref, x_ref, w1_ref, b1_ref, h_ref):
    xb = x_ref[0]
    h = jnp.dot(xb, w1_ref[0], preferred_element_type=jnp.float32)
    h_ref[0] = jax.nn.gelu(h + b1_ref[0, 0][None, :])


def _ffn2_body(eid_ref, perm_ref, w_ref, h_ref, w2_ref, b2_ref, out_ref):
    j = pl.program_id(1)
    hb = h_ref[0]
    o = jnp.dot(hb, w2_ref[0], preferred_element_type=jnp.float32)
    out_ref[0] = w_ref[j] * (o + b2_ref[0, 0][None, :])


def _ffn_pass1(eid_s, perm, x, W1, b1r):
    grid_spec = pltpu.PrefetchScalarGridSpec(
        num_scalar_prefetch=2,
        grid=(KF, B),
        in_specs=[
            pl.BlockSpec((1, T, D), lambda kf, j, eid, perm: (perm[j] // 2, 0, 0)),
            pl.BlockSpec((1, D, BF), lambda kf, j, eid, perm: (eid[j], 0, kf)),
            pl.BlockSpec((1, 1, BF), lambda kf, j, eid, perm: (eid[j], 0, kf)),
        ],
        out_specs=pl.BlockSpec((1, T, BF), lambda kf, j, eid, perm: (j, 0, kf)),
    )
    return pl.pallas_call(
        _ffn1_body,
        grid_spec=grid_spec,
        out_shape=jax.ShapeDtypeStruct((B, T, DFF), jnp.float32),
        compiler_params=pltpu.CompilerParams(
            dimension_semantics=("arbitrary", "arbitrary"),
        ),
    )(eid_s, perm, x, W1, b1r)


def _ffn_pass2(eid_s, perm, w_s, H, W2, b2r):
    grid_spec = pltpu.PrefetchScalarGridSpec(
        num_scalar_prefetch=3,
        grid=(KD, B),
        in_specs=[
            pl.BlockSpec((1, T, DFF), lambda kd, j, eid, perm, w: (j, 0, 0)),
            pl.BlockSpec((1, DFF, BD), lambda kd, j, eid, perm, w: (eid[j], 0, kd)),
            pl.BlockSpec((1, 1, BD), lambda kd, j, eid, perm, w: (eid[j], 0, kd)),
        ],
        out_specs=pl.BlockSpec((1, T, BD), lambda kd, j, eid, perm, w: (perm[j], 0, kd)),
    )
    return pl.pallas_call(
        _ffn2_body,
        grid_spec=grid_spec,
        out_shape=jax.ShapeDtypeStruct((B, T, D), jnp.float32),
        compiler_params=pltpu.CompilerParams(
            dimension_semantics=("arbitrary", "arbitrary"),
        ),
    )(eid_s, perm, w_s, H, W2, b2r)


@jax.jit
def kernel(x, Wg, W1, b1, W2, b2):
    x_avg = jnp.mean(x, axis=1)
    logits = x_avg @ Wg.T
    prob = jax.nn.softmax(logits, axis=-1)
    imp = jnp.sum(prob, axis=0)
    importance_loss = (jnp.std(imp, ddof=1) / jnp.mean(imp)) ** 2
    topv = jnp.max(prob, axis=-1)
    eid = jnp.argmax(prob, axis=-1).astype(jnp.int32)
    perm = jnp.argsort(eid, stable=True).astype(jnp.int32)
    eid_s = eid[perm]
    w_s = prob[perm // 2, eid_s]

    b1r = b1.reshape(NRE, 1, DFF)
    b2r = b2.reshape(NRE, 1, D)
    H = _ffn_pass1(eid_s, perm, x, W1, b1r)
    output = _ffn_pass2(eid_s, perm, w_s, H, W2, b2r)

    beam_scores = topv
    expert_route = eid[:, None]
    beam_idx = jnp.arange(B, dtype=jnp.int32)
    return (output, beam_scores, expert_route, beam_idx, importance_loss)


# speedup vs baseline: 2.4484x; 2.4484x over previous
import functools
import jax
import jax.numpy as jnp
from jax.experimental import pallas as pl
from jax.experimental.pallas import tpu as pltpu

B, T, D = 64, 32, 2048
NRE = 7
DFF = 2048
BF = 512
KF = DFF // BF
GS = 8
MG = GS * T
NG = 14


def _moe_body(ge_ref, grow_ref, gval_ref, gw_ref,
              x_ref, w1_ref, b1_ref, w2_ref, b2_ref,
              out_ref, xg_ref, acc_ref):
    g = pl.program_id(0)
    kf = pl.program_id(1)

    @pl.when(kf == 0)
    def _():
        for s in range(GS):
            xg_ref[s * T:(s + 1) * T] = x_ref[grow_ref[g, s] // 2]

    h = jnp.dot(xg_ref[...], w1_ref[0], preferred_element_type=jnp.float32)
    h = h + b1_ref[0, 0][None, :]
    gl = jax.nn.gelu(h)
    wcol = jnp.concatenate(
        [jnp.full((T, 1), gw_ref[g, s], jnp.float32) for s in range(GS)], axis=0)
    gl = gl * wcol
    contrib = jnp.dot(gl, w2_ref[0], preferred_element_type=jnp.float32)

    @pl.when(kf == 0)
    def _():
        acc_ref[...] = contrib

    @pl.when(kf > 0)
    def _():
        acc_ref[...] = acc_ref[...] + contrib

    @pl.when(kf == KF - 1)
    def _():
        total = acc_ref[...] + wcol * b2_ref[0, 0][None, :]
        for s in range(GS):
            @pl.when(gval_ref[g, s] > 0)
            def _():
                out_ref[grow_ref[g, s]] = total[s * T:(s + 1) * T]


def _moe_ffn(ge, grow, gval, gw, x, W1, b1r, W2, b2r):
    grid_spec = pltpu.PrefetchScalarGridSpec(
        num_scalar_prefetch=4,
        grid=(NG, KF),
        in_specs=[
            pl.BlockSpec((B, T, D), lambda g, kf, ge, gr, gv, gw: (0, 0, 0)),
            pl.BlockSpec((1, D, BF), lambda g, kf, ge, gr, gv, gw: (ge[g], 0, kf)),
            pl.BlockSpec((1, 1, BF), lambda g, kf, ge, gr, gv, gw: (ge[g], 0, kf)),
            pl.BlockSpec((1, BF, D), lambda g, kf, ge, gr, gv, gw: (ge[g], kf, 0)),
            pl.BlockSpec((1, 1, D), lambda g, kf, ge, gr, gv, gw: (ge[g], 0, 0)),
        ],
        out_specs=pl.BlockSpec((B, T, D), lambda g, kf, ge, gr, gv, gw: (0, 0, 0)),
        scratch_shapes=[
            pltpu.VMEM((MG, D), jnp.float32),
            pltpu.VMEM((MG, D), jnp.float32),
        ],
    )
    return pl.pallas_call(
        _moe_body,
        grid_spec=grid_spec,
        out_shape=jax.ShapeDtypeStruct((B, T, D), jnp.float32),
        compiler_params=pltpu.CompilerParams(
            dimension_semantics=("arbitrary", "arbitrary"),
            vmem_limit_bytes=110 * 1024 * 1024,
        ),
    )(ge, grow, gval, gw, x, W1, b1r, W2, b2r)


@jax.jit
def kernel(x, Wg, W1, b1, W2, b2):
    x_avg = jnp.mean(x, axis=1)
    logits = x_avg @ Wg.T
    prob = jax.nn.softmax(logits, axis=-1)
    imp = jnp.sum(prob, axis=0)
    importance_loss = (jnp.std(imp, ddof=1) / jnp.mean(imp)) ** 2
    topv = jnp.max(prob, axis=-1)
    eid = jnp.argmax(prob, axis=-1).astype(jnp.int32)

    perm = jnp.argsort(eid, stable=True).astype(jnp.int32)
    eid_s = eid[perm]
    counts = jnp.sum(eid[None, :] == jnp.arange(NRE, dtype=jnp.int32)[:, None],
                     axis=1).astype(jnp.int32)
    off = jnp.concatenate([jnp.zeros(1, jnp.int32), jnp.cumsum(counts)[:-1]])
    gpe = (counts + GS - 1) // GS
    gcum = jnp.cumsum(gpe)
    total_groups = gcum[-1]
    gids = jnp.arange(NG, dtype=jnp.int32)
    ge_raw = jnp.searchsorted(gcum, gids, side='right').astype(jnp.int32)
    valid_g = gids < total_groups
    ge = jnp.where(valid_g, ge_raw, 0).astype(jnp.int32)
    gi = gids - (gcum[ge] - gpe[ge])
    p0 = off[ge] + gi * GS
    pslots = p0[:, None] + jnp.arange(GS, dtype=jnp.int32)[None, :]
    gval = (pslots < (off[ge] + counts[ge])[:, None]) & valid_g[:, None]
    pclamp = jnp.minimum(pslots, B - 1)
    grow = perm[pclamp]
    gw = prob[grow // 2, ge[:, None]]
    gval = gval.astype(jnp.int32)

    b1r = b1.reshape(NRE, 1, DFF)
    b2r = b2.reshape(NRE, 1, D)
    output = _moe_ffn(ge, grow, gval, gw, x, W1, b1r, W2, b2r)

    beam_scores = topv
    expert_route = eid[:, None]
    beam_idx = jnp.arange(B, dtype=jnp.int32)
    return (output, beam_scores, expert_route, beam_idx, importance_loss)
